# Initial kernel scaffold; baseline (speedup 1.0000x reference)
import jax
import jax.numpy as jnp
from jax.experimental import pallas as pl


def kernel(h, edge_index, e, snorm_n, snorm_e, W0, b0, W1, b1, W2, b2):
    raise NotImplementedError("write your pallas kernel here")


# speedup vs baseline: 6.3488x; 6.3488x over previous
import jax
import jax.numpy as jnp
from jax import lax
from jax.experimental import pallas as pl
from jax.experimental.pallas import tpu as pltpu
from jax.experimental.pallas import tpu_sc as plsc

N = 10000
E = 320000
IN_DIM = 128
HID = 128
NC = 40
ALPHA = 0.1
K = 10

NT = 16
W = 48
ROWS_T = 640
NP = NT * ROWS_T
RB = 64
NB = ROWS_T // RB
CH = 128
NCH = 157
EPT = NCH * CH
MB = 1024


def _mlp_body(h_ref, w0_ref, b0_ref, w1_ref, b1_ref, w2_ref, b2_ref, o_ref):
    x = jnp.dot(h_ref[...], w0_ref[...], preferred_element_type=jnp.float32)
    x = jnp.maximum(x + b0_ref[...], 0.0)
    x = jnp.dot(x, w1_ref[...], preferred_element_type=jnp.float32)
    x = jnp.maximum(x + b1_ref[...], 0.0)
    o_ref[...] = (
        jnp.dot(x, w2_ref[...], preferred_element_type=jnp.float32) + b2_ref[...]
    )


def _mlp(hp, W0, b0, W1, b1, W2p, b2p):
    return pl.pallas_call(
        _mlp_body,
        grid=(NP // MB,),
        in_specs=[
            pl.BlockSpec((MB, IN_DIM), lambda i: (i, 0)),
            pl.BlockSpec((IN_DIM, HID), lambda i: (0, 0)),
            pl.BlockSpec((1, HID), lambda i: (0, 0)),
            pl.BlockSpec((HID, HID), lambda i: (0, 0)),
            pl.BlockSpec((1, HID), lambda i: (0, 0)),
            pl.BlockSpec((HID, W), lambda i: (0, 0)),
            pl.BlockSpec((1, W), lambda i: (0, 0)),
        ],
        out_specs=pl.BlockSpec((MB, W), lambda i: (i, 0)),
        out_shape=jax.ShapeDtypeStruct((NP, W), jnp.float32),
    )(hp, W0, b0, W1, b1, W2p, b2p)


def _rsqrt16(x):
    i = lax.bitcast_convert_type(x, jnp.int32)
    i = 1597463007 - lax.shift_right_logical(i, 1)
    y = lax.bitcast_convert_type(i, jnp.float32)
    for _ in range(3):
        y = y * (1.5 - 0.5 * x * y * y)
    return y


def _prop_body(x0_hbm, srcm_hbm, dstm_hbm, out_hbm, z_hbm,
               agg_sh, src_v, dst_v, hz_v, c1w_v, abuf, gbuf, zeros_v,
               sem):
    wid = lax.axis_index("s")
    base = wid * ROWS_T

    pltpu.sync_copy(srcm_hbm.at[wid], src_v)
    pltpu.sync_copy(dstm_hbm.at[wid], dst_v)
    pltpu.sync_copy(x0_hbm.at[pl.ds(base, ROWS_T)], hz_v)

    zeros16 = jnp.zeros((16,), jnp.float32)
    ones16 = jnp.ones((16,), jnp.float32)

    def init_zeros(r, c):
        for j in range(W // 16):
            zeros_v[r, pl.ds(j * 16, 16)] = zeros16
        return c
    lax.fori_loop(0, RB, init_zeros, 0)

    def init_ones(r, c):
        for j in range(W // 16):
            gbuf[r, pl.ds(j * 16, 16)] = ones16
        return c
    lax.fori_loop(0, CH, init_ones, 0)

    def zero_blk(b, c):
        pltpu.sync_copy(zeros_v, agg_sh.at[pl.ds(base + b * RB, RB)])
        return c
    lax.fori_loop(0, NB, zero_blk, 0)
    plsc.subcore_barrier()

    def deg_chunk(j, c):
        pltpu.sync_copy(gbuf, agg_sh.at[dst_v.at[j]], add=True)
        return c
    lax.fori_loop(0, NCH, deg_chunk, 0)
    plsc.subcore_barrier()

    def prep_blk(b, c):
        rbase = base + b * RB
        pltpu.sync_copy(agg_sh.at[pl.ds(rbase, RB)], abuf)

        def prep_row(r, c2):
            rl = b * RB + r
            deg = abuf[r, pl.ds(0, 16)]
            dc = jnp.maximum(deg, 1.0)
            nrm = _rsqrt16(dc)
            rowid = jnp.zeros((16,), jnp.int32) + (rbase + r)
            validf = jnp.clip((N - rowid).astype(jnp.float32), 0.0, 1.0)
            nrm = nrm * validf
            c1 = validf * ((1.0 - ALPHA) / dc)
            c1w_v[rl, pl.ds(0, 16)] = c1
            for j in range(W // 16):
                sl = pl.ds(j * 16, 16)
                x0v = hz_v[rl, sl]
                abuf[r, sl] = nrm * x0v
                hz_v[rl, sl] = ALPHA * nrm * x0v
            return c2
        lax.fori_loop(0, RB, prep_row, 0)
        pltpu.sync_copy(abuf, z_hbm.at[pl.ds(rbase, RB)])
        pltpu.sync_copy(zeros_v, agg_sh.at[pl.ds(rbase, RB)])
        return c
    lax.fori_loop(0, NB, prep_blk, 0)
    plsc.subcore_barrier()

    def round_body(k, c):
        def chunk(j, c2):
            pltpu.async_copy(z_hbm.at[src_v.at[j]], gbuf, sem).wait()
            pltpu.sync_copy(gbuf, agg_sh.at[dst_v.at[j]], add=True)
            return c2
        lax.fori_loop(0, NCH, chunk, 0)
        plsc.subcore_barrier()

        def upd_blk(b, c2):
            rbase = base + b * RB
            pltpu.sync_copy(agg_sh.at[pl.ds(rbase, RB)], abuf)

            def upd_row(r, c3):
                rl = b * RB + r
                c1 = c1w_v[rl, pl.ds(0, 16)]
                for j in range(W // 16):
                    sl = pl.ds(j * 16, 16)
                    abuf[r, sl] = c1 * abuf[r, sl] + hz_v[rl, sl]
                return c3
            lax.fori_loop(0, RB, upd_row, 0)
            pltpu.sync_copy(abuf, z_hbm.at[pl.ds(rbase, RB)])
            pltpu.sync_copy(zeros_v, agg_sh.at[pl.ds(rbase, RB)])
            return c2
        lax.fori_loop(0, NB, upd_blk, 0)
        plsc.subcore_barrier()
        return c
    lax.fori_loop(0, K, round_body, 0)

    def fin_blk(b, c):
        rbase = base + b * RB
        pltpu.sync_copy(z_hbm.at[pl.ds(rbase, RB)], abuf)

        def fin_row(r, c2):
            rl = b * RB + r
            rn = _rsqrt16(c1w_v[rl, pl.ds(0, 16)] * (1.0 / (1.0 - ALPHA)))
            for j in range(W // 16):
                sl = pl.ds(j * 16, 16)
                abuf[r, sl] = abuf[r, sl] * rn
            return c2
        lax.fori_loop(0, RB, fin_row, 0)
        pltpu.sync_copy(abuf, out_hbm.at[pl.ds(rbase, RB)])
        return c
    lax.fori_loop(0, NB, fin_blk, 0)


_prop = pl.kernel(
    _prop_body,
    out_type=(jax.ShapeDtypeStruct((NP, W), jnp.float32),
              jax.ShapeDtypeStruct((NP, W), jnp.float32)),
    mesh=plsc.VectorSubcoreMesh(
        core_axis_name="c", subcore_axis_name="s",
        num_cores=1, num_subcores=NT),
    compiler_params=pltpu.CompilerParams(use_tc_tiling_on_sc=False),
    scratch_types=[
        pltpu.VMEM_SHARED((NP, W), jnp.float32),
        pltpu.VMEM((NCH, CH), jnp.int32),
        pltpu.VMEM((NCH, CH), jnp.int32),
        pltpu.VMEM((ROWS_T, W), jnp.float32),
        pltpu.VMEM((ROWS_T, 16), jnp.float32),
        pltpu.VMEM((RB, W), jnp.float32),
        pltpu.VMEM((CH, W), jnp.float32),
        pltpu.VMEM((RB, W), jnp.float32),
        pltpu.SemaphoreType.DMA,
    ],
)


def kernel(h, edge_index, e, snorm_n, snorm_e, W0, b0, W1, b1, W2, b2):
    del e, snorm_n, snorm_e
    hp = jnp.pad(h, ((0, NP - N), (0, 0)))
    W2p = jnp.pad(W2, ((0, 0), (0, W - NC)))
    b2p = jnp.pad(b2, ((0, W - NC)))
    x0 = _mlp(hp, W0, b0.reshape(1, HID), W1, b1.reshape(1, HID),
              W2p, b2p.reshape(1, W))

    pad = NT * EPT - E
    pad_idx = N + (jnp.arange(pad, dtype=jnp.int32) % (NP - N))
    srcm = jnp.concatenate([edge_index[0], pad_idx]).reshape(NT, NCH, CH)
    dstm = jnp.concatenate([edge_index[1], pad_idx]).reshape(NT, NCH, CH)

    out, _ = _prop(x0, srcm, dstm)
    return out[:N, :NC]


# speedup vs baseline: 7.9681x; 1.2551x over previous
import jax
import jax.numpy as jnp
from jax import lax
from jax.experimental import pallas as pl
from jax.experimental.pallas import tpu as pltpu
from jax.experimental.pallas import tpu_sc as plsc

N = 10000
E = 320000
IN_DIM = 128
HID = 128
NC = 40
ALPHA = 0.1
K = 10

NT = 16
W = 48
ROWS_T = 640
NP = NT * ROWS_T
RB = 64
NB = ROWS_T // RB
CH = 128
NCH = 160
G = 20
NGR = NCH // G
EPT = NCH * CH
MB = 1024


def _mlp_body(h_ref, w0_ref, b0_ref, w1_ref, b1_ref, w2_ref, b2_ref, o_ref):
    x = jnp.dot(h_ref[...], w0_ref[...], preferred_element_type=jnp.float32)
    x = jnp.maximum(x + b0_ref[...], 0.0)
    x = jnp.dot(x, w1_ref[...], preferred_element_type=jnp.float32)
    x = jnp.maximum(x + b1_ref[...], 0.0)
    o_ref[...] = (
        jnp.dot(x, w2_ref[...], preferred_element_type=jnp.float32) + b2_ref[...]
    )


def _mlp(hp, W0, b0, W1, b1, W2p, b2p):
    return pl.pallas_call(
        _mlp_body,
        grid=(NP // MB,),
        in_specs=[
            pl.BlockSpec((MB, IN_DIM), lambda i: (i, 0)),
            pl.BlockSpec((IN_DIM, HID), lambda i: (0, 0)),
            pl.BlockSpec((1, HID), lambda i: (0, 0)),
            pl.BlockSpec((HID, HID), lambda i: (0, 0)),
            pl.BlockSpec((1, HID), lambda i: (0, 0)),
            pl.BlockSpec((HID, W), lambda i: (0, 0)),
            pl.BlockSpec((1, W), lambda i: (0, 0)),
        ],
        out_specs=pl.BlockSpec((MB, W), lambda i: (i, 0)),
        out_shape=jax.ShapeDtypeStruct((NP, W), jnp.float32),
    )(hp, W0, b0, W1, b1, W2p, b2p)


def _rsqrt16(x):
    i = lax.bitcast_convert_type(x, jnp.int32)
    i = 1597463007 - lax.shift_right_logical(i, 1)
    y = lax.bitcast_convert_type(i, jnp.float32)
    for _ in range(3):
        y = y * (1.5 - 0.5 * x * y * y)
    return y


def _prop_body(x0_hbm, srcm_hbm, dstm_hbm, out_hbm,
               z_sh, agg_sh, src_v, dst_v, hz_v, c1w_v, abuf, gbuf, zeros_v,
               isem):
    wid = lax.axis_index("s")
    base = wid * ROWS_T
    gbase = wid * NGR

    pltpu.sync_copy(x0_hbm.at[pl.ds(base, ROWS_T)], hz_v)

    zeros16 = jnp.zeros((16,), jnp.float32)
    ones16 = jnp.ones((16,), jnp.float32)

    def init_zeros(r, c):
        for j in range(W // 16):
            zeros_v[r, pl.ds(j * 16, 16)] = zeros16
        return c
    lax.fori_loop(0, RB, init_zeros, 0)

    def init_ones(r, c):
        for j in range(W // 16):
            gbuf[r, pl.ds(j * 16, 16)] = ones16
        return c
    lax.fori_loop(0, CH, init_ones, 0)

    def zero_blk(b, c):
        pltpu.sync_copy(zeros_v, agg_sh.at[pl.ds(base + b * RB, RB)])
        return c
    lax.fori_loop(0, NB, zero_blk, 0)
    plsc.subcore_barrier()

    def edge_pass(use_gather):
        cps = [pltpu.async_copy(srcm_hbm.at[gbase], src_v.at[pl.ds(0, G)],
                                isem),
               pltpu.async_copy(dstm_hbm.at[gbase], dst_v.at[pl.ds(0, G)],
                                isem)]
        for g in range(NGR):
            for cp in cps:
                cp.wait()
            if g + 1 < NGR:
                p2 = ((g + 1) % 2) * G
                cps = [pltpu.async_copy(srcm_hbm.at[gbase + g + 1],
                                        src_v.at[pl.ds(p2, G)], isem),
                       pltpu.async_copy(dstm_hbm.at[gbase + g + 1],
                                        dst_v.at[pl.ds(p2, G)], isem)]
            pb = (g % 2) * G

            def chunk(j, c, pb=pb):
                row = pb + j
                if use_gather:
                    pltpu.sync_copy(z_sh.at[src_v.at[row]], gbuf)
                pltpu.sync_copy(gbuf, agg_sh.at[dst_v.at[row]], add=True)
                return c
            lax.fori_loop(0, G, chunk, 0)

    edge_pass(use_gather=False)
    plsc.subcore_barrier()

    def prep_blk(b, c):
        rbase = base + b * RB
        pltpu.sync_copy(agg_sh.at[pl.ds(rbase, RB)], abuf)

        def prep_row(r, c2):
            rl = b * RB + r
            deg = abuf[r, pl.ds(0, 16)]
            dc = jnp.maximum(deg, 1.0)
            nrm = _rsqrt16(dc)
            rowid = jnp.zeros((16,), jnp.int32) + (rbase + r)
            validf = jnp.clip((N - rowid).astype(jnp.float32), 0.0, 1.0)
            nrm = nrm * validf
            c1 = validf * ((1.0 - ALPHA) / dc)
            c1w_v[rl, pl.ds(0, 16)] = c1
            for j in range(W // 16):
                sl = pl.ds(j * 16, 16)
                x0v = hz_v[rl, sl]
                abuf[r, sl] = nrm * x0v
                hz_v[rl, sl] = ALPHA * nrm * x0v
            return c2
        lax.fori_loop(0, RB, prep_row, 0)
        pltpu.sync_copy(abuf, z_sh.at[pl.ds(rbase, RB)])
        pltpu.sync_copy(zeros_v, agg_sh.at[pl.ds(rbase, RB)])
        return c
    lax.fori_loop(0, NB, prep_blk, 0)
    plsc.subcore_barrier()

    def round_body(k, c):
        edge_pass(use_gather=True)
        plsc.subcore_barrier()

        def upd_blk(b, c2):
            rbase = base + b * RB
            pltpu.sync_copy(agg_sh.at[pl.ds(rbase, RB)], abuf)

            def upd_row(r, c3):
                rl = b * RB + r
                c1 = c1w_v[rl, pl.ds(0, 16)]
                for j in range(W // 16):
                    sl = pl.ds(j * 16, 16)
                    abuf[r, sl] = c1 * abuf[r, sl] + hz_v[rl, sl]
                return c3
            lax.fori_loop(0, RB, upd_row, 0)
            pltpu.sync_copy(abuf, z_sh.at[pl.ds(rbase, RB)])
            pltpu.sync_copy(zeros_v, agg_sh.at[pl.ds(rbase, RB)])
            return c2
        lax.fori_loop(0, NB, upd_blk, 0)
        plsc.subcore_barrier()
        return c
    lax.fori_loop(0, K, round_body, 0)

    def fin_blk(b, c):
        rbase = base + b * RB
        pltpu.sync_copy(z_sh.at[pl.ds(rbase, RB)], abuf)

        def fin_row(r, c2):
            rl = b * RB + r
            rn = _rsqrt16(c1w_v[rl, pl.ds(0, 16)] * (1.0 / (1.0 - ALPHA)))
            for j in range(W // 16):
                sl = pl.ds(j * 16, 16)
                abuf[r, sl] = abuf[r, sl] * rn
            return c2
        lax.fori_loop(0, RB, fin_row, 0)
        pltpu.sync_copy(abuf, out_hbm.at[pl.ds(rbase, RB)])
        return c
    lax.fori_loop(0, NB, fin_blk, 0)


_prop = pl.kernel(
    _prop_body,
    out_type=jax.ShapeDtypeStruct((NP, W), jnp.float32),
    mesh=plsc.VectorSubcoreMesh(
        core_axis_name="c", subcore_axis_name="s",
        num_cores=1, num_subcores=NT),
    compiler_params=pltpu.CompilerParams(use_tc_tiling_on_sc=False),
    scratch_types=[
        pltpu.VMEM_SHARED((NP, W), jnp.float32),
        pltpu.VMEM_SHARED((NP, W), jnp.float32),
        pltpu.VMEM((2 * G, CH), jnp.int32),
        pltpu.VMEM((2 * G, CH), jnp.int32),
        pltpu.VMEM((ROWS_T, W), jnp.float32),
        pltpu.VMEM((ROWS_T, 16), jnp.float32),
        pltpu.VMEM((RB, W), jnp.float32),
        pltpu.VMEM((CH, W), jnp.float32),
        pltpu.VMEM((RB, W), jnp.float32),
        pltpu.SemaphoreType.DMA,
    ],
)


def kernel(h, edge_index, e, snorm_n, snorm_e, W0, b0, W1, b1, W2, b2):
    del e, snorm_n, snorm_e
    hp = jnp.pad(h, ((0, NP - N), (0, 0)))
    W2p = jnp.pad(W2, ((0, 0), (0, W - NC)))
    b2p = jnp.pad(b2, ((0, W - NC)))
    x0 = _mlp(hp, W0, b0.reshape(1, HID), W1, b1.reshape(1, HID),
              W2p, b2p.reshape(1, W))

    pad = NT * EPT - E
    pad_idx = N + (jnp.arange(pad, dtype=jnp.int32) % (NP - N))
    srcm = jnp.concatenate([edge_index[0], pad_idx]).reshape(NT * NGR, G, CH)
    dstm = jnp.concatenate([edge_index[1], pad_idx]).reshape(NT * NGR, G, CH)

    return _prop(x0, srcm, dstm)[:N, :NC]


# speedup vs baseline: 10.2519x; 1.2866x over previous
import jax
import jax.numpy as jnp
from jax import lax
from jax.experimental import pallas as pl
from jax.experimental.pallas import tpu as pltpu
from jax.experimental.pallas import tpu_sc as plsc

N = 10000
E = 320000
IN_DIM = 128
HID = 128
NC = 40
ALPHA = 0.1
K = 10

NT = 16
W = 48
ROWS_T = 640
NP = NT * ROWS_T
RB = 64
NB = ROWS_T // RB
CH = 128
NCH = 160
G = 16
NGR = NCH // G
EPT = NCH * CH
MB = 1024


def _mlp_body(h_ref, w0_ref, b0_ref, w1_ref, b1_ref, w2_ref, b2_ref, o_ref):
    x = jnp.dot(h_ref[...], w0_ref[...], preferred_element_type=jnp.float32)
    x = jnp.maximum(x + b0_ref[...], 0.0)
    x = jnp.dot(x, w1_ref[...], preferred_element_type=jnp.float32)
    x = jnp.maximum(x + b1_ref[...], 0.0)
    o_ref[...] = (
        jnp.dot(x, w2_ref[...], preferred_element_type=jnp.float32) + b2_ref[...]
    )


def _mlp(hp, W0, b0, W1, b1, W2p, b2p):
    return pl.pallas_call(
        _mlp_body,
        grid=(NP // MB,),
        in_specs=[
            pl.BlockSpec((MB, IN_DIM), lambda i: (i, 0)),
            pl.BlockSpec((IN_DIM, HID), lambda i: (0, 0)),
            pl.BlockSpec((1, HID), lambda i: (0, 0)),
            pl.BlockSpec((HID, HID), lambda i: (0, 0)),
            pl.BlockSpec((1, HID), lambda i: (0, 0)),
            pl.BlockSpec((HID, W), lambda i: (0, 0)),
            pl.BlockSpec((1, W), lambda i: (0, 0)),
        ],
        out_specs=pl.BlockSpec((MB, W), lambda i: (i, 0)),
        out_shape=jax.ShapeDtypeStruct((NP, W), jnp.float32),
    )(hp, W0, b0, W1, b1, W2p, b2p)


def _rsqrt16(x):
    i = lax.bitcast_convert_type(x, jnp.int32)
    i = 1597463007 - lax.shift_right_logical(i, 1)
    y = lax.bitcast_convert_type(i, jnp.float32)
    for _ in range(3):
        y = y * (1.5 - 0.5 * x * y * y)
    return y


def _prop_body(x0_hbm, srcm_hbm, dstm_hbm, out_hbm,
               z_sh, agg_sh, src_v, dst_v, hz_v, c1w_v, abuf, gbuf, zeros_v,
               isem, gsem):
    wid = lax.axis_index("s")
    base = wid * ROWS_T
    gbase = wid * NGR

    pltpu.sync_copy(x0_hbm.at[pl.ds(base, ROWS_T)], hz_v)

    zeros16 = jnp.zeros((16,), jnp.float32)
    ones16 = jnp.ones((16,), jnp.float32)

    def init_zeros(r, c):
        for j in range(W // 16):
            zeros_v[r, pl.ds(j * 16, 16)] = zeros16
        return c
    lax.fori_loop(0, RB, init_zeros, 0)

    def init_ones(r, c):
        for j in range(W // 16):
            gbuf[0, r, pl.ds(j * 16, 16)] = ones16
        return c
    lax.fori_loop(0, CH, init_ones, 0)

    def zero_blk(b, c):
        pltpu.sync_copy(zeros_v, agg_sh.at[pl.ds(base + b * RB, RB)])
        return c
    lax.fori_loop(0, NB, zero_blk, 0)
    plsc.subcore_barrier()

    def _gwait(s):
        pltpu.make_async_copy(x0_hbm.at[pl.ds(0, CH)], gbuf.at[s],
                              gsem).wait()

    def _fire_idx(g, par):
        return [pltpu.async_copy(srcm_hbm.at[gbase + g],
                                 src_v.at[pl.ds(par * G, G)], isem),
                pltpu.async_copy(dstm_hbm.at[gbase + g],
                                 dst_v.at[pl.ds(par * G, G)], isem)]

    def edge_pass(use_gather):
        cps = _fire_idx(0, 0)
        for g in range(NGR):
            for cp in cps:
                cp.wait()
            if g + 1 < NGR:
                cps = _fire_idx(g + 1, (g + 1) % 2)
            pb = (g % 2) * G

            if not use_gather:
                def chunk(j, c, pb=pb):
                    pltpu.sync_copy(gbuf.at[0], agg_sh.at[dst_v.at[pb + j]],
                                    add=True)
                    return c
                lax.fori_loop(0, G, chunk, 0)
                continue

            pltpu.async_copy(z_sh.at[src_v.at[pb]], gbuf.at[0], gsem)

            def pair(t, c, pb=pb):
                r0 = pb + 2 * t
                _gwait(0)
                pltpu.async_copy(z_sh.at[src_v.at[r0 + 1]], gbuf.at[1], gsem)
                pltpu.sync_copy(gbuf.at[0], agg_sh.at[dst_v.at[r0]],
                                add=True)
                _gwait(1)
                pltpu.async_copy(z_sh.at[src_v.at[r0 + 2]], gbuf.at[0], gsem)
                pltpu.sync_copy(gbuf.at[1], agg_sh.at[dst_v.at[r0 + 1]],
                                add=True)
                return c
            lax.fori_loop(0, G // 2 - 1, pair, 0)

            _gwait(0)
            pltpu.async_copy(z_sh.at[src_v.at[pb + G - 1]], gbuf.at[1], gsem)
            pltpu.sync_copy(gbuf.at[0], agg_sh.at[dst_v.at[pb + G - 2]],
                            add=True)
            _gwait(1)
            pltpu.sync_copy(gbuf.at[1], agg_sh.at[dst_v.at[pb + G - 1]],
                            add=True)

    edge_pass(use_gather=False)
    plsc.subcore_barrier()

    def prep_blk(b, c):
        rbase = base + b * RB
        pltpu.sync_copy(agg_sh.at[pl.ds(rbase, RB)], abuf)

        def prep_row(r, c2):
            rl = b * RB + r
            deg = abuf[r, pl.ds(0, 16)]
            dc = jnp.maximum(deg, 1.0)
            nrm = _rsqrt16(dc)
            rowid = jnp.zeros((16,), jnp.int32) + (rbase + r)
            validf = jnp.clip((N - rowid).astype(jnp.float32), 0.0, 1.0)
            nrm = nrm * validf
            c1 = validf * ((1.0 - ALPHA) / dc)
            c1w_v[rl, pl.ds(0, 16)] = c1
            for j in range(W // 16):
                sl = pl.ds(j * 16, 16)
                x0v = hz_v[rl, sl]
                abuf[r, sl] = nrm * x0v
                hz_v[rl, sl] = ALPHA * nrm * x0v
            return c2
        lax.fori_loop(0, RB, prep_row, 0)
        pltpu.sync_copy(abuf, z_sh.at[pl.ds(rbase, RB)])
        pltpu.sync_copy(zeros_v, agg_sh.at[pl.ds(rbase, RB)])
        return c
    lax.fori_loop(0, NB, prep_blk, 0)
    plsc.subcore_barrier()

    def round_body(k, c):
        edge_pass(use_gather=True)
        plsc.subcore_barrier()

        def upd_blk(b, c2):
            rbase = base + b * RB
            pltpu.sync_copy(agg_sh.at[pl.ds(rbase, RB)], abuf)

            def upd_row(r, c3):
                rl = b * RB + r
                c1 = c1w_v[rl, pl.ds(0, 16)]
                for j in range(W // 16):
                    sl = pl.ds(j * 16, 16)
                    abuf[r, sl] = c1 * abuf[r, sl] + hz_v[rl, sl]
                return c3
            lax.fori_loop(0, RB, upd_row, 0)
            pltpu.sync_copy(abuf, z_sh.at[pl.ds(rbase, RB)])
            pltpu.sync_copy(zeros_v, agg_sh.at[pl.ds(rbase, RB)])
            return c2
        lax.fori_loop(0, NB, upd_blk, 0)
        plsc.subcore_barrier()
        return c
    lax.fori_loop(0, K, round_body, 0)

    def fin_blk(b, c):
        rbase = base + b * RB
        pltpu.sync_copy(z_sh.at[pl.ds(rbase, RB)], abuf)

        def fin_row(r, c2):
            rl = b * RB + r
            rn = _rsqrt16(c1w_v[rl, pl.ds(0, 16)] * (1.0 / (1.0 - ALPHA)))
            for j in range(W // 16):
                sl = pl.ds(j * 16, 16)
                abuf[r, sl] = abuf[r, sl] * rn
            return c2
        lax.fori_loop(0, RB, fin_row, 0)
        pltpu.sync_copy(abuf, out_hbm.at[pl.ds(rbase, RB)])
        return c
    lax.fori_loop(0, NB, fin_blk, 0)


_prop = pl.kernel(
    _prop_body,
    out_type=jax.ShapeDtypeStruct((NP, W), jnp.float32),
    mesh=plsc.VectorSubcoreMesh(
        core_axis_name="c", subcore_axis_name="s",
        num_cores=1, num_subcores=NT),
    compiler_params=pltpu.CompilerParams(use_tc_tiling_on_sc=False),
    scratch_types=[
        pltpu.VMEM_SHARED((NP, W), jnp.float32),
        pltpu.VMEM_SHARED((NP, W), jnp.float32),
        pltpu.VMEM((2 * G, CH), jnp.int32),
        pltpu.VMEM((2 * G, CH), jnp.int32),
        pltpu.VMEM((ROWS_T, W), jnp.float32),
        pltpu.VMEM((ROWS_T, 16), jnp.float32),
        pltpu.VMEM((RB, W), jnp.float32),
        pltpu.VMEM((2, CH, W), jnp.float32),
        pltpu.VMEM((RB, W), jnp.float32),
        pltpu.SemaphoreType.DMA,
        pltpu.SemaphoreType.DMA,
    ],
)


def kernel(h, edge_index, e, snorm_n, snorm_e, W0, b0, W1, b1, W2, b2):
    del e, snorm_n, snorm_e
    hp = jnp.pad(h, ((0, NP - N), (0, 0)))
    W2p = jnp.pad(W2, ((0, 0), (0, W - NC)))
    b2p = jnp.pad(b2, ((0, W - NC)))
    x0 = _mlp(hp, W0, b0.reshape(1, HID), W1, b1.reshape(1, HID),
              W2p, b2p.reshape(1, W))

    pad = NT * EPT - E
    pad_idx = N + (jnp.arange(pad, dtype=jnp.int32) % (NP - N))
    srcm = jnp.concatenate([edge_index[0], pad_idx]).reshape(NT * NGR, G, CH)
    dstm = jnp.concatenate([edge_index[1], pad_idx]).reshape(NT * NGR, G, CH)

    return _prop(x0, srcm, dstm)[:N, :NC]


# speedup vs baseline: 18.9758x; 1.8510x over previous
import jax
import jax.numpy as jnp
from jax import lax
from jax.experimental import pallas as pl
from jax.experimental.pallas import tpu as pltpu
from jax.experimental.pallas import tpu_sc as plsc

N = 10000
E = 320000
IN_DIM = 128
HID = 128
NC = 40
ALPHA = 0.1
K = 10

NCORE = 2
NT = 16
W = 48
WC = W // NCORE
ROWS_T = 640
NP = NT * ROWS_T
RB = 64
NB = ROWS_T // RB
CH = 128
NCH = 160
EPT = NCH * CH
MB = 1024

SL0, SL1 = 0, WC - 16


def _mlp_body(h_ref, w0_ref, b0_ref, w1_ref, b1_ref, w2_ref, b2_ref, o_ref):
    x = jnp.dot(h_ref[...], w0_ref[...], preferred_element_type=jnp.float32)
    x = jnp.maximum(x + b0_ref[...], 0.0)
    x = jnp.dot(x, w1_ref[...], preferred_element_type=jnp.float32)
    x = jnp.maximum(x + b1_ref[...], 0.0)
    o_ref[...] = (
        jnp.dot(x, w2_ref[...], preferred_element_type=jnp.float32) + b2_ref[...]
    )


def _mlp(hp, W0, b0, W1, b1, W2p, b2p):
    return pl.pallas_call(
        _mlp_body,
        grid=(NP // MB,),
        in_specs=[
            pl.BlockSpec((MB, IN_DIM), lambda i: (i, 0)),
            pl.BlockSpec((IN_DIM, HID), lambda i: (0, 0)),
            pl.BlockSpec((1, HID), lambda i: (0, 0)),
            pl.BlockSpec((HID, HID), lambda i: (0, 0)),
            pl.BlockSpec((1, HID), lambda i: (0, 0)),
            pl.BlockSpec((HID, W), lambda i: (0, 0)),
            pl.BlockSpec((1, W), lambda i: (0, 0)),
        ],
        out_specs=pl.BlockSpec((MB, W), lambda i: (i, 0)),
        out_shape=jax.ShapeDtypeStruct((NP, W), jnp.float32),
    )(hp, W0, b0, W1, b1, W2p, b2p)


def _rsqrt16(x):
    i = lax.bitcast_convert_type(x, jnp.int32)
    i = 1597463007 - lax.shift_right_logical(i, 1)
    y = lax.bitcast_convert_type(i, jnp.float32)
    for _ in range(3):
        y = y * (1.5 - 0.5 * x * y * y)
    return y


def _prop_body(x0_hbm, srcm_hbm, dstm_hbm, out_hbm,
               z_sh, agg_sh, src_v, dst_v, hz_v, c1w_v, abuf, gbuf, zeros_v,
               gsem):
    cid = lax.axis_index("c")
    sid = lax.axis_index("s")
    base = sid * ROWS_T
    hbase = cid * NP + base

    pltpu.sync_copy(srcm_hbm.at[sid], src_v)
    pltpu.sync_copy(dstm_hbm.at[sid], dst_v)
    pltpu.sync_copy(x0_hbm.at[pl.ds(hbase, ROWS_T)], hz_v)

    zeros16 = jnp.zeros((16,), jnp.float32)
    ones16 = jnp.ones((16,), jnp.float32)

    def init_zeros(r, c):
        zeros_v[r, pl.ds(SL0, 16)] = zeros16
        zeros_v[r, pl.ds(SL1, 16)] = zeros16
        return c
    lax.fori_loop(0, RB, init_zeros, 0)

    def init_ones(r, c):
        gbuf[0, r, pl.ds(SL0, 16)] = ones16
        gbuf[0, r, pl.ds(SL1, 16)] = ones16
        return c
    lax.fori_loop(0, CH, init_ones, 0)

    def zero_blk(b, c):
        pltpu.sync_copy(zeros_v, agg_sh.at[pl.ds(base + b * RB, RB)])
        return c
    lax.fori_loop(0, NB, zero_blk, 0)
    plsc.subcore_barrier()

    def deg_chunk(j, c):
        pltpu.sync_copy(gbuf.at[0], agg_sh.at[dst_v.at[j]], add=True)
        return c
    lax.fori_loop(0, NCH, deg_chunk, 0)
    plsc.subcore_barrier()

    def prep_blk(b, c):
        rbase = base + b * RB
        pltpu.sync_copy(agg_sh.at[pl.ds(rbase, RB)], abuf)

        def prep_row(r, c2):
            rl = b * RB + r
            deg = abuf[r, pl.ds(0, 16)]
            dc = jnp.maximum(deg, 1.0)
            nrm = _rsqrt16(dc)
            rowid = jnp.zeros((16,), jnp.int32) + (rbase + r)
            validf = jnp.clip((N - rowid).astype(jnp.float32), 0.0, 1.0)
            nrm = nrm * validf
            c1 = validf * ((1.0 - ALPHA) / dc)
            c1w_v[rl, pl.ds(0, 16)] = c1
            x00 = hz_v[rl, pl.ds(SL0, 16)]
            x01 = hz_v[rl, pl.ds(SL1, 16)]
            abuf[r, pl.ds(SL0, 16)] = nrm * x00
            abuf[r, pl.ds(SL1, 16)] = nrm * x01
            hz_v[rl, pl.ds(SL0, 16)] = ALPHA * nrm * x00
            hz_v[rl, pl.ds(SL1, 16)] = ALPHA * nrm * x01
            return c2
        lax.fori_loop(0, RB, prep_row, 0)
        pltpu.sync_copy(abuf, z_sh.at[pl.ds(rbase, RB)])
        pltpu.sync_copy(zeros_v, agg_sh.at[pl.ds(rbase, RB)])
        return c
    lax.fori_loop(0, NB, prep_blk, 0)
    plsc.subcore_barrier()

    def _gwait(s):
        pltpu.make_async_copy(x0_hbm.at[pl.ds(0, CH)], gbuf.at[s],
                              gsem).wait()

    def round_body(k, c):
        pltpu.async_copy(z_sh.at[src_v.at[0]], gbuf.at[0], gsem)

        def pair(t, c2):
            r0 = 2 * t
            _gwait(0)
            pltpu.async_copy(z_sh.at[src_v.at[r0 + 1]], gbuf.at[1], gsem)
            pltpu.sync_copy(gbuf.at[0], agg_sh.at[dst_v.at[r0]], add=True)
            _gwait(1)
            pltpu.async_copy(z_sh.at[src_v.at[r0 + 2]], gbuf.at[0], gsem)
            pltpu.sync_copy(gbuf.at[1], agg_sh.at[dst_v.at[r0 + 1]],
                            add=True)
            return c2
        lax.fori_loop(0, NCH // 2 - 1, pair, 0)

        _gwait(0)
        pltpu.async_copy(z_sh.at[src_v.at[NCH - 1]], gbuf.at[1], gsem)
        pltpu.sync_copy(gbuf.at[0], agg_sh.at[dst_v.at[NCH - 2]], add=True)
        _gwait(1)
        pltpu.sync_copy(gbuf.at[1], agg_sh.at[dst_v.at[NCH - 1]], add=True)
        plsc.subcore_barrier()

        def upd_blk(b, c2):
            rbase = base + b * RB
            pltpu.sync_copy(agg_sh.at[pl.ds(rbase, RB)], abuf)

            def upd_row(r, c3):
                rl = b * RB + r
                c1 = c1w_v[rl, pl.ds(0, 16)]
                a0 = abuf[r, pl.ds(SL0, 16)]
                a1 = abuf[r, pl.ds(SL1, 16)]
                abuf[r, pl.ds(SL0, 16)] = c1 * a0 + hz_v[rl, pl.ds(SL0, 16)]
                abuf[r, pl.ds(SL1, 16)] = c1 * a1 + hz_v[rl, pl.ds(SL1, 16)]
                return c3
            lax.fori_loop(0, RB, upd_row, 0)
            pltpu.sync_copy(abuf, z_sh.at[pl.ds(rbase, RB)])
            pltpu.sync_copy(zeros_v, agg_sh.at[pl.ds(rbase, RB)])
            return c2
        lax.fori_loop(0, NB, upd_blk, 0)
        plsc.subcore_barrier()
        return c
    lax.fori_loop(0, K, round_body, 0)

    def fin_blk(b, c):
        rbase = base + b * RB
        pltpu.sync_copy(z_sh.at[pl.ds(rbase, RB)], abuf)

        def fin_row(r, c2):
            rl = b * RB + r
            rn = _rsqrt16(c1w_v[rl, pl.ds(0, 16)] * (1.0 / (1.0 - ALPHA)))
            z0 = abuf[r, pl.ds(SL0, 16)]
            z1 = abuf[r, pl.ds(SL1, 16)]
            abuf[r, pl.ds(SL0, 16)] = z0 * rn
            abuf[r, pl.ds(SL1, 16)] = z1 * rn
            return c2
        lax.fori_loop(0, RB, fin_row, 0)
        pltpu.sync_copy(abuf, out_hbm.at[pl.ds(cid * NP + rbase, RB)])
        return c
    lax.fori_loop(0, NB, fin_blk, 0)


_prop = pl.kernel(
    _prop_body,
    out_type=jax.ShapeDtypeStruct((NCORE * NP, WC), jnp.float32),
    mesh=plsc.VectorSubcoreMesh(
        core_axis_name="c", subcore_axis_name="s",
        num_cores=NCORE, num_subcores=NT),
    compiler_params=pltpu.CompilerParams(use_tc_tiling_on_sc=False),
    scratch_types=[
        pltpu.VMEM_SHARED((NP, WC), jnp.float32),
        pltpu.VMEM_SHARED((NP, WC), jnp.float32),
        pltpu.VMEM((NCH, CH), jnp.int32),
        pltpu.VMEM((NCH, CH), jnp.int32),
        pltpu.VMEM((ROWS_T, WC), jnp.float32),
        pltpu.VMEM((ROWS_T, 16), jnp.float32),
        pltpu.VMEM((RB, WC), jnp.float32),
        pltpu.VMEM((2, CH, WC), jnp.float32),
        pltpu.VMEM((RB, WC), jnp.float32),
        pltpu.SemaphoreType.DMA,
    ],
)


def kernel(h, edge_index, e, snorm_n, snorm_e, W0, b0, W1, b1, W2, b2):
    del e, snorm_n, snorm_e
    hp = jnp.pad(h, ((0, NP - N), (0, 0)))
    W2p = jnp.pad(W2, ((0, 0), (0, W - NC)))
    b2p = jnp.pad(b2, ((0, W - NC)))
    x0 = _mlp(hp, W0, b0.reshape(1, HID), W1, b1.reshape(1, HID),
              W2p, b2p.reshape(1, W))
    x0c = x0.reshape(NP, NCORE, WC).transpose(1, 0, 2).reshape(NCORE * NP, WC)

    pad = NT * EPT - E
    pad_idx = N + (jnp.arange(pad, dtype=jnp.int32) % (NP - N))
    srcm = jnp.concatenate([edge_index[0], pad_idx]).reshape(NT, NCH, CH)
    dstm = jnp.concatenate([edge_index[1], pad_idx]).reshape(NT, NCH, CH)

    outc = _prop(x0c, srcm, dstm)
    out = outc.reshape(NCORE, NP, WC).transpose(1, 0, 2).reshape(NP, W)
    return out[:N, :NC]


# speedup vs baseline: 19.4061x; 1.0227x over previous
import jax
import jax.numpy as jnp
from jax import lax
from jax.experimental import pallas as pl
from jax.experimental.pallas import tpu as pltpu
from jax.experimental.pallas import tpu_sc as plsc

N = 10000
E = 320000
IN_DIM = 128
HID = 128
NC = 40
ALPHA = 0.1
K = 10

NCORE = 2
NT = 16
W = 48
WC = W // NCORE
ROWS_T = 640
NP = NT * ROWS_T
RB = 64
NB = ROWS_T // RB
CH = 128
NCH = 160
EPT = NCH * CH
MB = 1024

SL0, SL1 = 0, WC - 16


def _mlp_body(h_ref, w0_ref, b0_ref, w1_ref, b1_ref, w2_ref, b2_ref, o_ref):
    x = jnp.dot(h_ref[...], w0_ref[...], preferred_element_type=jnp.float32)
    x = jnp.maximum(x + b0_ref[...], 0.0)
    x = jnp.dot(x, w1_ref[...], preferred_element_type=jnp.float32)
    x = jnp.maximum(x + b1_ref[...], 0.0)
    o_ref[...] = (
        jnp.dot(x, w2_ref[...], preferred_element_type=jnp.float32) + b2_ref[...]
    )


def _mlp(hp, W0, b0, W1, b1, W2p, b2p):
    return pl.pallas_call(
        _mlp_body,
        grid=(NP // MB,),
        in_specs=[
            pl.BlockSpec((MB, IN_DIM), lambda i: (i, 0)),
            pl.BlockSpec((IN_DIM, HID), lambda i: (0, 0)),
            pl.BlockSpec((1, HID), lambda i: (0, 0)),
            pl.BlockSpec((HID, HID), lambda i: (0, 0)),
            pl.BlockSpec((1, HID), lambda i: (0, 0)),
            pl.BlockSpec((HID, W), lambda i: (0, 0)),
            pl.BlockSpec((1, W), lambda i: (0, 0)),
        ],
        out_specs=pl.BlockSpec((MB, W), lambda i: (i, 0)),
        out_shape=jax.ShapeDtypeStruct((NP, W), jnp.float32),
    )(hp, W0, b0, W1, b1, W2p, b2p)


NCH2 = NCH // NCORE


def _deg_body(dstm_hbm, deg_hbm, agg_sh, dst_v, ones_v, zeros_v):
    cid = lax.axis_index("c")
    sid = lax.axis_index("s")
    base = sid * ROWS_T

    pltpu.sync_copy(dstm_hbm.at[sid, pl.ds(cid * NCH2, NCH2)], dst_v)

    zeros16 = jnp.zeros((16,), jnp.float32)
    ones16 = jnp.ones((16,), jnp.float32)

    def init_bufs(r, c):
        zeros_v[r, pl.ds(0, 16)] = zeros16
        return c
    lax.fori_loop(0, RB, init_bufs, 0)

    def init_ones(r, c):
        ones_v[r, pl.ds(0, 16)] = ones16
        return c
    lax.fori_loop(0, CH, init_ones, 0)

    def zero_blk(b, c):
        pltpu.sync_copy(zeros_v, agg_sh.at[pl.ds(base + b * RB, RB)])
        return c
    lax.fori_loop(0, NB, zero_blk, 0)
    plsc.subcore_barrier()

    def deg_chunk(j, c):
        pltpu.sync_copy(ones_v, agg_sh.at[dst_v.at[j]], add=True)
        return c
    lax.fori_loop(0, NCH2, deg_chunk, 0)
    plsc.subcore_barrier()

    def out_blk(b, c):
        pltpu.sync_copy(agg_sh.at[pl.ds(base + b * RB, RB)],
                        deg_hbm.at[pl.ds(cid * NP + base + b * RB, RB)])
        return c
    lax.fori_loop(0, NB, out_blk, 0)


_deg = pl.kernel(
    _deg_body,
    out_type=jax.ShapeDtypeStruct((NCORE * NP, 16), jnp.float32),
    mesh=plsc.VectorSubcoreMesh(
        core_axis_name="c", subcore_axis_name="s",
        num_cores=NCORE, num_subcores=NT),
    compiler_params=pltpu.CompilerParams(use_tc_tiling_on_sc=False),
    scratch_types=[
        pltpu.VMEM_SHARED((NP, 16), jnp.float32),
        pltpu.VMEM((NCH2, CH), jnp.int32),
        pltpu.VMEM((CH, 16), jnp.float32),
        pltpu.VMEM((RB, 16), jnp.float32),
    ],
)


def _rsqrt16(x):
    i = lax.bitcast_convert_type(x, jnp.int32)
    i = 1597463007 - lax.shift_right_logical(i, 1)
    y = lax.bitcast_convert_type(i, jnp.float32)
    for _ in range(3):
        y = y * (1.5 - 0.5 * x * y * y)
    return y


def _prop_body(x0_hbm, degp_hbm, srcm_hbm, dstm_hbm, out_hbm,
               z_sh, agg_sh, src_v, dst_v, hz_v, c1w_v, abuf, gbuf, zeros_v,
               dbuf0, dbuf1, gsem):
    cid = lax.axis_index("c")
    sid = lax.axis_index("s")
    base = sid * ROWS_T
    hbase = cid * NP + base

    pltpu.sync_copy(srcm_hbm.at[sid], src_v)
    pltpu.sync_copy(dstm_hbm.at[sid], dst_v)
    pltpu.sync_copy(x0_hbm.at[pl.ds(hbase, ROWS_T)], hz_v)

    zeros16 = jnp.zeros((16,), jnp.float32)

    def init_zeros(r, c):
        zeros_v[r, pl.ds(SL0, 16)] = zeros16
        zeros_v[r, pl.ds(SL1, 16)] = zeros16
        return c
    lax.fori_loop(0, RB, init_zeros, 0)

    def zero_blk(b, c):
        pltpu.sync_copy(zeros_v, agg_sh.at[pl.ds(base + b * RB, RB)])
        return c
    lax.fori_loop(0, NB, zero_blk, 0)

    def prep_blk(b, c):
        rbase = base + b * RB
        pltpu.sync_copy(degp_hbm.at[pl.ds(rbase, RB)], dbuf0)
        pltpu.sync_copy(degp_hbm.at[pl.ds(NP + rbase, RB)], dbuf1)

        def prep_row(r, c2):
            rl = b * RB + r
            deg = dbuf0[r, pl.ds(0, 16)] + dbuf1[r, pl.ds(0, 16)]
            dc = jnp.maximum(deg, 1.0)
            nrm = _rsqrt16(dc)
            rowid = jnp.zeros((16,), jnp.int32) + (rbase + r)
            validf = jnp.clip((N - rowid).astype(jnp.float32), 0.0, 1.0)
            nrm = nrm * validf
            c1 = validf * ((1.0 - ALPHA) / dc)
            c1w_v[rl, pl.ds(0, 16)] = c1
            x00 = hz_v[rl, pl.ds(SL0, 16)]
            x01 = hz_v[rl, pl.ds(SL1, 16)]
            abuf[r, pl.ds(SL0, 16)] = nrm * x00
            abuf[r, pl.ds(SL1, 16)] = nrm * x01
            hz_v[rl, pl.ds(SL0, 16)] = ALPHA * nrm * x00
            hz_v[rl, pl.ds(SL1, 16)] = ALPHA * nrm * x01
            return c2
        lax.fori_loop(0, RB, prep_row, 0)
        pltpu.sync_copy(abuf, z_sh.at[pl.ds(rbase, RB)])
        return c
    lax.fori_loop(0, NB, prep_blk, 0)
    plsc.subcore_barrier()

    def _gwait(s):
        pltpu.make_async_copy(x0_hbm.at[pl.ds(0, CH)], gbuf.at[s],
                              gsem).wait()

    def round_body(k, c):
        pltpu.async_copy(z_sh.at[src_v.at[0]], gbuf.at[0], gsem)

        def pair(t, c2):
            r0 = 2 * t
            _gwait(0)
            pltpu.async_copy(z_sh.at[src_v.at[r0 + 1]], gbuf.at[1], gsem)
            pltpu.sync_copy(gbuf.at[0], agg_sh.at[dst_v.at[r0]], add=True)
            _gwait(1)
            pltpu.async_copy(z_sh.at[src_v.at[r0 + 2]], gbuf.at[0], gsem)
            pltpu.sync_copy(gbuf.at[1], agg_sh.at[dst_v.at[r0 + 1]],
                            add=True)
            return c2
        lax.fori_loop(0, NCH // 2 - 1, pair, 0)

        _gwait(0)
        pltpu.async_copy(z_sh.at[src_v.at[NCH - 1]], gbuf.at[1], gsem)
        pltpu.sync_copy(gbuf.at[0], agg_sh.at[dst_v.at[NCH - 2]], add=True)
        _gwait(1)
        pltpu.sync_copy(gbuf.at[1], agg_sh.at[dst_v.at[NCH - 1]], add=True)
        plsc.subcore_barrier()

        def upd_blk(b, c2):
            rbase = base + b * RB
            pltpu.sync_copy(agg_sh.at[pl.ds(rbase, RB)], abuf)

            def upd_row(r, c3):
                rl = b * RB + r
                c1 = c1w_v[rl, pl.ds(0, 16)]
                a0 = abuf[r, pl.ds(SL0, 16)]
                a1 = abuf[r, pl.ds(SL1, 16)]
                abuf[r, pl.ds(SL0, 16)] = c1 * a0 + hz_v[rl, pl.ds(SL0, 16)]
                abuf[r, pl.ds(SL1, 16)] = c1 * a1 + hz_v[rl, pl.ds(SL1, 16)]
                return c3
            lax.fori_loop(0, RB, upd_row, 0)
            pltpu.sync_copy(abuf, z_sh.at[pl.ds(rbase, RB)])
            pltpu.sync_copy(zeros_v, agg_sh.at[pl.ds(rbase, RB)])
            return c2
        lax.fori_loop(0, NB, upd_blk, 0)
        plsc.subcore_barrier()
        return c
    lax.fori_loop(0, K, round_body, 0)

    def fin_blk(b, c):
        rbase = base + b * RB
        pltpu.sync_copy(z_sh.at[pl.ds(rbase, RB)], abuf)

        def fin_row(r, c2):
            rl = b * RB + r
            rn = _rsqrt16(c1w_v[rl, pl.ds(0, 16)] * (1.0 / (1.0 - ALPHA)))
            z0 = abuf[r, pl.ds(SL0, 16)]
            z1 = abuf[r, pl.ds(SL1, 16)]
            abuf[r, pl.ds(SL0, 16)] = z0 * rn
            abuf[r, pl.ds(SL1, 16)] = z1 * rn
            return c2
        lax.fori_loop(0, RB, fin_row, 0)
        pltpu.sync_copy(abuf, out_hbm.at[pl.ds(cid * NP + rbase, RB)])
        return c
    lax.fori_loop(0, NB, fin_blk, 0)


_prop = pl.kernel(
    _prop_body,
    out_type=jax.ShapeDtypeStruct((NCORE * NP, WC), jnp.float32),
    mesh=plsc.VectorSubcoreMesh(
        core_axis_name="c", subcore_axis_name="s",
        num_cores=NCORE, num_subcores=NT),
    compiler_params=pltpu.CompilerParams(use_tc_tiling_on_sc=False),
    scratch_types=[
        pltpu.VMEM_SHARED((NP, WC), jnp.float32),
        pltpu.VMEM_SHARED((NP, WC), jnp.float32),
        pltpu.VMEM((NCH, CH), jnp.int32),
        pltpu.VMEM((NCH, CH), jnp.int32),
        pltpu.VMEM((ROWS_T, WC), jnp.float32),
        pltpu.VMEM((ROWS_T, 16), jnp.float32),
        pltpu.VMEM((RB, WC), jnp.float32),
        pltpu.VMEM((2, CH, WC), jnp.float32),
        pltpu.VMEM((RB, WC), jnp.float32),
        pltpu.VMEM((RB, 16), jnp.float32),
        pltpu.VMEM((RB, 16), jnp.float32),
        pltpu.SemaphoreType.DMA,
    ],
)


def kernel(h, edge_index, e, snorm_n, snorm_e, W0, b0, W1, b1, W2, b2):
    del e, snorm_n, snorm_e
    hp = jnp.pad(h, ((0, NP - N), (0, 0)))
    W2p = jnp.pad(W2, ((0, 0), (0, W - NC)))
    b2p = jnp.pad(b2, ((0, W - NC)))
    x0 = _mlp(hp, W0, b0.reshape(1, HID), W1, b1.reshape(1, HID),
              W2p, b2p.reshape(1, W))
    x0c = x0.reshape(NP, NCORE, WC).transpose(1, 0, 2).reshape(NCORE * NP, WC)

    pad = NT * EPT - E
    pad_idx = N + (jnp.arange(pad, dtype=jnp.int32) % (NP - N))
    srcm = jnp.concatenate([edge_index[0], pad_idx]).reshape(NT, NCH, CH)
    dstm = jnp.concatenate([edge_index[1], pad_idx]).reshape(NT, NCH, CH)

    degp = _deg(dstm)
    outc = _prop(x0c, degp, srcm, dstm)
    out = outc.reshape(NCORE, NP, WC).transpose(1, 0, 2).reshape(NP, W)
    return out[:N, :NC]


# speedup vs baseline: 19.7463x; 1.0175x over previous
import jax
import jax.numpy as jnp
from jax import lax
from jax.experimental import pallas as pl
from jax.experimental.pallas import tpu as pltpu
from jax.experimental.pallas import tpu_sc as plsc

N = 10000
E = 320000
IN_DIM = 128
HID = 128
NC = 40
ALPHA = 0.1
K = 10

NCORE = 2
NT = 16
W = 48
WC = W // NCORE
ROWS_T = 640
NP = NT * ROWS_T
RB = 64
NB = ROWS_T // RB
CH = 128
NCH = 160
EPT = NCH * CH
MB = 1024

SL0, SL1 = 0, WC - 16


def _mlp_body(h_ref, w0_ref, b0_ref, w1_ref, b1_ref, w2_ref, b2_ref, o_ref):
    x = jnp.dot(h_ref[...], w0_ref[...], preferred_element_type=jnp.float32)
    x = jnp.maximum(x + b0_ref[...], 0.0)
    x = jnp.dot(x, w1_ref[...], preferred_element_type=jnp.float32)
    x = jnp.maximum(x + b1_ref[...], 0.0)
    o_ref[...] = (
        jnp.dot(x, w2_ref[...], preferred_element_type=jnp.float32) + b2_ref[...]
    )


def _mlp(hp, W0, b0, W1, b1, W2p, b2p):
    return pl.pallas_call(
        _mlp_body,
        grid=(NP // MB,),
        in_specs=[
            pl.BlockSpec((MB, IN_DIM), lambda i: (i, 0)),
            pl.BlockSpec((IN_DIM, HID), lambda i: (0, 0)),
            pl.BlockSpec((1, HID), lambda i: (0, 0)),
            pl.BlockSpec((HID, HID), lambda i: (0, 0)),
            pl.BlockSpec((1, HID), lambda i: (0, 0)),
            pl.BlockSpec((HID, W), lambda i: (0, 0)),
            pl.BlockSpec((1, W), lambda i: (0, 0)),
        ],
        out_specs=pl.BlockSpec((MB, W), lambda i: (i, 0)),
        out_shape=jax.ShapeDtypeStruct((NP, W), jnp.float32),
    )(hp, W0, b0, W1, b1, W2p, b2p)


NCH2 = NCH // NCORE


def _deg_body(dstm_hbm, deg_hbm, agg_sh, dst_v, ones_v, zeros_v):
    cid = lax.axis_index("c")
    sid = lax.axis_index("s")
    base = sid * ROWS_T

    pltpu.sync_copy(dstm_hbm.at[sid, pl.ds(cid * NCH2, NCH2)], dst_v)

    zeros16 = jnp.zeros((16,), jnp.float32)
    ones16 = jnp.ones((16,), jnp.float32)

    def init_bufs(r, c):
        zeros_v[r, pl.ds(0, 16)] = zeros16
        return c
    lax.fori_loop(0, RB, init_bufs, 0)

    def init_ones(r, c):
        ones_v[r, pl.ds(0, 16)] = ones16
        return c
    lax.fori_loop(0, CH, init_ones, 0)

    def zero_blk(b, c):
        pltpu.sync_copy(zeros_v, agg_sh.at[pl.ds(base + b * RB, RB)])
        return c
    lax.fori_loop(0, NB, zero_blk, 0)
    plsc.subcore_barrier()

    def deg_chunk(j, c):
        pltpu.sync_copy(ones_v, agg_sh.at[dst_v.at[j]], add=True)
        return c
    lax.fori_loop(0, NCH2, deg_chunk, 0)
    plsc.subcore_barrier()

    def out_blk(b, c):
        pltpu.sync_copy(agg_sh.at[pl.ds(base + b * RB, RB)],
                        deg_hbm.at[pl.ds(cid * NP + base + b * RB, RB)])
        return c
    lax.fori_loop(0, NB, out_blk, 0)


_deg = pl.kernel(
    _deg_body,
    out_type=jax.ShapeDtypeStruct((NCORE * NP, 16), jnp.float32),
    mesh=plsc.VectorSubcoreMesh(
        core_axis_name="c", subcore_axis_name="s",
        num_cores=NCORE, num_subcores=NT),
    compiler_params=pltpu.CompilerParams(use_tc_tiling_on_sc=False),
    scratch_types=[
        pltpu.VMEM_SHARED((NP, 16), jnp.float32),
        pltpu.VMEM((NCH2, CH), jnp.int32),
        pltpu.VMEM((CH, 16), jnp.float32),
        pltpu.VMEM((RB, 16), jnp.float32),
    ],
)


def _rsqrt16(x):
    i = lax.bitcast_convert_type(x, jnp.int32)
    i = 1597463007 - lax.shift_right_logical(i, 1)
    y = lax.bitcast_convert_type(i, jnp.float32)
    for _ in range(3):
        y = y * (1.5 - 0.5 * x * y * y)
    return y


def _prop_body(x0_hbm, degp_hbm, srcm_hbm, dstm_hbm, out_hbm,
               z_sh, agg_sh, src_v, dst_v, hz_v, c1w_v, abuf, gbuf, zeros_v,
               dbuf0, dbuf1, gsem):
    cid = lax.axis_index("c")
    sid = lax.axis_index("s")
    base = sid * ROWS_T
    hbase = cid * NP + base

    pltpu.sync_copy(srcm_hbm.at[sid], src_v)
    pltpu.sync_copy(dstm_hbm.at[sid], dst_v)
    pltpu.sync_copy(x0_hbm.at[pl.ds(hbase, ROWS_T)], hz_v)

    zeros16 = jnp.zeros((16,), jnp.float32)

    def init_zeros(r, c):
        zeros_v[r, pl.ds(SL0, 16)] = zeros16
        zeros_v[r, pl.ds(SL1, 16)] = zeros16
        return c
    lax.fori_loop(0, RB, init_zeros, 0)

    def zero_blk(b, c):
        pltpu.sync_copy(zeros_v, agg_sh.at[pl.ds(base + b * RB, RB)])
        return c
    lax.fori_loop(0, NB, zero_blk, 0)

    def prep_blk(b, c):
        rbase = base + b * RB
        pltpu.sync_copy(degp_hbm.at[pl.ds(rbase, RB)], dbuf0)
        pltpu.sync_copy(degp_hbm.at[pl.ds(NP + rbase, RB)], dbuf1)

        def prep_row(r, c2):
            rl = b * RB + r
            deg = dbuf0[r, pl.ds(0, 16)] + dbuf1[r, pl.ds(0, 16)]
            dc = jnp.maximum(deg, 1.0)
            nrm = _rsqrt16(dc)
            rowid = jnp.zeros((16,), jnp.int32) + (rbase + r)
            validf = jnp.clip((N - rowid).astype(jnp.float32), 0.0, 1.0)
            nrm = nrm * validf
            c1 = validf * ((1.0 - ALPHA) / dc)
            c1w_v[rl, pl.ds(0, 16)] = c1
            x00 = hz_v[rl, pl.ds(SL0, 16)]
            x01 = hz_v[rl, pl.ds(SL1, 16)]
            abuf[r, pl.ds(SL0, 16)] = nrm * x00
            abuf[r, pl.ds(SL1, 16)] = nrm * x01
            hz_v[rl, pl.ds(SL0, 16)] = ALPHA * nrm * x00
            hz_v[rl, pl.ds(SL1, 16)] = ALPHA * nrm * x01
            return c2
        lax.fori_loop(0, RB, prep_row, 0)
        pltpu.sync_copy(abuf, z_sh.at[pl.ds(rbase, RB)])
        return c
    lax.fori_loop(0, NB, prep_blk, 0)
    plsc.subcore_barrier()

    def _gwait(s):
        pltpu.make_async_copy(x0_hbm.at[pl.ds(0, CH)], gbuf.at[s],
                              gsem).wait()

    def edge_pass():
        pltpu.async_copy(z_sh.at[src_v.at[0]], gbuf.at[0], gsem)

        def pair(t, c2):
            r0 = 2 * t
            _gwait(0)
            pltpu.async_copy(z_sh.at[src_v.at[r0 + 1]], gbuf.at[1], gsem)
            pltpu.sync_copy(gbuf.at[0], agg_sh.at[dst_v.at[r0]], add=True)
            _gwait(1)
            pltpu.async_copy(z_sh.at[src_v.at[r0 + 2]], gbuf.at[0], gsem)
            pltpu.sync_copy(gbuf.at[1], agg_sh.at[dst_v.at[r0 + 1]],
                            add=True)
            return c2
        lax.fori_loop(0, NCH // 2 - 1, pair, 0)

        _gwait(0)
        pltpu.async_copy(z_sh.at[src_v.at[NCH - 1]], gbuf.at[1], gsem)
        pltpu.sync_copy(gbuf.at[0], agg_sh.at[dst_v.at[NCH - 2]], add=True)
        _gwait(1)
        pltpu.sync_copy(gbuf.at[1], agg_sh.at[dst_v.at[NCH - 1]], add=True)
        plsc.subcore_barrier()

    def round_body(k, c):
        edge_pass()

        def upd_blk(b, c2):
            rbase = base + b * RB
            pltpu.sync_copy(agg_sh.at[pl.ds(rbase, RB)], abuf)

            def upd_row(r, c3):
                rl = b * RB + r
                c1 = c1w_v[rl, pl.ds(0, 16)]
                a0 = abuf[r, pl.ds(SL0, 16)]
                a1 = abuf[r, pl.ds(SL1, 16)]
                abuf[r, pl.ds(SL0, 16)] = c1 * a0 + hz_v[rl, pl.ds(SL0, 16)]
                abuf[r, pl.ds(SL1, 16)] = c1 * a1 + hz_v[rl, pl.ds(SL1, 16)]
                return c3
            lax.fori_loop(0, RB, upd_row, 0)
            pltpu.sync_copy(abuf, z_sh.at[pl.ds(rbase, RB)])
            pltpu.sync_copy(zeros_v, agg_sh.at[pl.ds(rbase, RB)])
            return c2
        lax.fori_loop(0, NB, upd_blk, 0)
        plsc.subcore_barrier()
        return c
    lax.fori_loop(0, K - 1, round_body, 0)

    edge_pass()

    def fin_blk(b, c):
        rbase = base + b * RB
        pltpu.sync_copy(agg_sh.at[pl.ds(rbase, RB)], abuf)

        def fin_row(r, c2):
            rl = b * RB + r
            c1 = c1w_v[rl, pl.ds(0, 16)]
            rn = _rsqrt16(c1 * (1.0 / (1.0 - ALPHA)))
            a0 = abuf[r, pl.ds(SL0, 16)]
            a1 = abuf[r, pl.ds(SL1, 16)]
            z0 = c1 * a0 + hz_v[rl, pl.ds(SL0, 16)]
            z1 = c1 * a1 + hz_v[rl, pl.ds(SL1, 16)]
            abuf[r, pl.ds(SL0, 16)] = z0 * rn
            abuf[r, pl.ds(SL1, 16)] = z1 * rn
            return c2
        lax.fori_loop(0, RB, fin_row, 0)
        pltpu.sync_copy(abuf, out_hbm.at[pl.ds(cid * NP + rbase, RB)])
        return c
    lax.fori_loop(0, NB, fin_blk, 0)


_prop = pl.kernel(
    _prop_body,
    out_type=jax.ShapeDtypeStruct((NCORE * NP, WC), jnp.float32),
    mesh=plsc.VectorSubcoreMesh(
        core_axis_name="c", subcore_axis_name="s",
        num_cores=NCORE, num_subcores=NT),
    compiler_params=pltpu.CompilerParams(use_tc_tiling_on_sc=False),
    scratch_types=[
        pltpu.VMEM_SHARED((NP, WC), jnp.float32),
        pltpu.VMEM_SHARED((NP, WC), jnp.float32),
        pltpu.VMEM((NCH, CH), jnp.int32),
        pltpu.VMEM((NCH, CH), jnp.int32),
        pltpu.VMEM((ROWS_T, WC), jnp.float32),
        pltpu.VMEM((ROWS_T, 16), jnp.float32),
        pltpu.VMEM((RB, WC), jnp.float32),
        pltpu.VMEM((2, CH, WC), jnp.float32),
        pltpu.VMEM((RB, WC), jnp.float32),
        pltpu.VMEM((RB, 16), jnp.float32),
        pltpu.VMEM((RB, 16), jnp.float32),
        pltpu.SemaphoreType.DMA,
    ],
)


def kernel(h, edge_index, e, snorm_n, snorm_e, W0, b0, W1, b1, W2, b2):
    del e, snorm_n, snorm_e
    hp = jnp.pad(h, ((0, NP - N), (0, 0)))
    W2p = jnp.pad(W2, ((0, 0), (0, W - NC)))
    b2p = jnp.pad(b2, ((0, W - NC)))
    x0 = _mlp(hp, W0, b0.reshape(1, HID), W1, b1.reshape(1, HID),
              W2p, b2p.reshape(1, W))
    x0c = x0.reshape(NP, NCORE, WC).transpose(1, 0, 2).reshape(NCORE * NP, WC)

    pad = NT * EPT - E
    pad_idx = N + (jnp.arange(pad, dtype=jnp.int32) % (NP - N))
    srcm = jnp.concatenate([edge_index[0], pad_idx]).reshape(NT, NCH, CH)
    dstm = jnp.concatenate([edge_index[1], pad_idx]).reshape(NT, NCH, CH)

    degp = _deg(dstm)
    outc = _prop(x0c, degp, srcm, dstm)
    out = outc.reshape(NCORE, NP, WC).transpose(1, 0, 2).reshape(NP, W)
    return out[:N, :NC]
